# Initial kernel scaffold; baseline (speedup 1.0000x reference)
import jax
import jax.numpy as jnp
from jax.experimental import pallas as pl


def kernel(x, edge_index, batch, W1, b1, W2, b2, W3, b3, Wl, bl):
    raise NotImplementedError("write your pallas kernel here")


# speedup vs baseline: 2.3166x; 2.3166x over previous
import jax
import jax.numpy as jnp
from jax.experimental import pallas as pl

N = 100000
G = 1024


def _gcn_conv(x, edge_index, W, b, dinv):
    xw = x @ W
    y = xw * dinv[:, None]
    acc = jax.ops.segment_sum(y[edge_index[0]], edge_index[1], num_segments=N)
    return (acc + y) * dinv[:, None] + b


def _final_body(p_ref, w_ref, b_ref, o_ref):
    o_ref[...] = p_ref[...] @ w_ref[...] + b_ref[0, 0]


def kernel(x, edge_index, batch, W1, b1, W2, b2, W3, b3, Wl, bl):
    ones = jnp.ones((edge_index.shape[1],), jnp.float32)
    deg = jax.ops.segment_sum(ones, edge_index[1], num_segments=N) + 1.0
    dinv = jax.lax.rsqrt(deg)
    h = jax.nn.relu(_gcn_conv(x, edge_index, W1, b1, dinv))
    h = jax.nn.relu(_gcn_conv(h, edge_index, W2, b2, dinv))
    h = jax.nn.relu(_gcn_conv(h, edge_index, W3, b3, dinv))
    s = jax.ops.segment_sum(h, batch, num_segments=G)
    cnt = jax.ops.segment_sum(jnp.ones((N,), h.dtype), batch, num_segments=G)
    pooled = s / jnp.clip(cnt, 1.0)[:, None]
    out = pl.pallas_call(
        _final_body,
        out_shape=jax.ShapeDtypeStruct((G, 1), jnp.float32),
    )(pooled, Wl, bl.reshape(1, 1))
    return out


# speedup vs baseline: 7.4507x; 3.2163x over previous
import functools

import jax
import jax.numpy as jnp
from jax import lax
from jax.experimental import pallas as pl
from jax.experimental.pallas import tpu as pltpu
from jax.experimental.pallas import tpu_sc as plsc

N = 100000
E = 1600000
H = 64
G = 1024

NP2 = 100352
BK = 8
BS = 12544
SAC = BS
ACC_R = 14336
PT_E = E // 32
GRP = 512
MAXG = (PT_E + GRP - 1) // GRP
SL = NP2 // 16
CH = 10000

_mesh = plsc.VectorSubcoreMesh(core_axis_name="c", subcore_axis_name="s")
_sc_params = pltpu.CompilerParams(
    needs_layout_passes=False, use_tc_tiling_on_sc=False)


@functools.partial(
    pl.kernel, mesh=_mesh,
    out_type=jax.ShapeDtypeStruct((32, NP2), jnp.float32),
    scratch_types=[
        pltpu.VMEM((NP2,), jnp.float32),
        pltpu.VMEM((CH,), jnp.int32),
    ],
    compiler_params=_sc_params,
)
def _deg_kernel(dst_hbm, out_hbm, partial, dstbuf):
    cid = lax.axis_index("c")
    sid = lax.axis_index("s")
    wid = cid * 16 + sid

    zeros = jnp.zeros((16,), jnp.float32)
    def zbody(i, _):
        partial[pl.ds(i * 16, 16)] = zeros
        return 0
    lax.fori_loop(0, NP2 // 16, zbody, 0)

    ones = jnp.ones((16,), jnp.float32)
    ebase = wid * PT_E
    def chunk_body(k, _):
        pltpu.sync_copy(dst_hbm.at[pl.ds(ebase + k * CH, CH)], dstbuf)
        def vbody(i, _):
            idx = dstbuf[pl.ds(i * 16, 16)]
            plsc.addupdate_scatter(partial, [idx], ones)
            return 0
        lax.fori_loop(0, CH // 16, vbody, 0)
        return 0
    lax.fori_loop(0, PT_E // CH, chunk_body, 0)
    pltpu.sync_copy(partial, out_hbm.at[wid])


@functools.partial(
    pl.kernel, mesh=_mesh,
    out_type=(
        jax.ShapeDtypeStruct((32, BK, MAXG, 4, 128), jnp.int32),
        jax.ShapeDtypeStruct((32, BK, MAXG, 4, 128), jnp.int32),
        jax.ShapeDtypeStruct((32, 32), jnp.int32),
    ),
    scratch_types=[
        pltpu.VMEM((CH,), jnp.int32),
        pltpu.VMEM((CH,), jnp.int32),
        pltpu.VMEM((BK * 1024,), jnp.int32),
        pltpu.VMEM((BK * 1024,), jnp.int32),
        pltpu.VMEM((32,), jnp.int32),
    ],
    compiler_params=_sc_params,
)
def _bin_kernel(src_hbm, dst_hbm, bsrc_hbm, bdst_hbm, ngrp_hbm,
                sbuf, dbuf, sstg, dstg, cntw):
    cid = lax.axis_index("c")
    sid = lax.axis_index("s")
    t = cid * 16 + sid
    ebase = t * PT_E

    def flush(b, g):
        for j in range(4):
            pltpu.sync_copy(sstg.at[pl.ds(b * 1024 + j * 128, 128)],
                            bsrc_hbm.at[t].at[b].at[g].at[j])
            pltpu.sync_copy(dstg.at[pl.ds(b * 1024 + j * 128, 128)],
                            bdst_hbm.at[t].at[b].at[g].at[j])

    def chunk_body(c, carry):
        pltpu.sync_copy(src_hbm.at[pl.ds(ebase + c * CH, CH)], sbuf)
        pltpu.sync_copy(dst_hbm.at[pl.ds(ebase + c * CH, CH)], dbuf)

        def sub_body(s, carry):
            def vreg_body(v, carry):
                fills, ngs = carry
                off = (s * 25 + v) * 16
                sv = sbuf[pl.ds(off, 16)]
                dv = dbuf[pl.ds(off, 16)]
                nf = []
                for b in range(BK):
                    msk = (dv >= b * BS) & (dv < (b + 1) * BS)
                    plsc.store_compressed(
                        sstg.at[pl.ds(b * 1024 + fills[b], 16)], sv, mask=msk)
                    plsc.store_compressed(
                        dstg.at[pl.ds(b * 1024 + fills[b], 16)],
                        dv - b * BS, mask=msk)
                    cnt = plsc.all_reduce_population_count(msk)[0]
                    nf.append(fills[b] + cnt)
                return tuple(nf), ngs
            carry = lax.fori_loop(0, 25, vreg_body, carry)
            fills, ngs = carry
            nf, nn = [], []
            for b in range(BK):
                def do_flush(fb, gb, b=b):
                    flush(b, gb)
                    for k in range(25):
                        sstg[pl.ds(b * 1024 + k * 16, 16)] = (
                            sstg[pl.ds(b * 1024 + 512 + k * 16, 16)])
                        dstg[pl.ds(b * 1024 + k * 16, 16)] = (
                            dstg[pl.ds(b * 1024 + 512 + k * 16, 16)])
                    return fb - 512, gb + 1
                fb, gb = lax.cond(
                    fills[b] >= 512, do_flush,
                    lambda fb, gb: (fb, gb), fills[b], ngs[b])
                nf.append(fb)
                nn.append(gb)
            return tuple(nf), tuple(nn)
        return lax.fori_loop(0, 25, sub_body, carry)

    z8 = (jnp.int32(0),) * BK
    fills, ngs = lax.fori_loop(0, PT_E // CH, chunk_body, (z8, z8))

    zv = jnp.zeros((16,), jnp.int32)
    sacv = jnp.full((16,), SAC, jnp.int32)
    nfinal = []
    for b in range(BK):
        for k in range(32):
            sstg[pl.ds(b * 1024 + fills[b] + k * 16, 16)] = zv
            dstg[pl.ds(b * 1024 + fills[b] + k * 16, 16)] = sacv
        def do_final(gb, b=b):
            flush(b, gb)
            return gb + 1
        gb = lax.cond(fills[b] > 0, do_final, lambda gb: gb, ngs[b])
        nfinal.append(gb)

    lane = lax.iota(jnp.int32, 16)
    v = jnp.zeros((16,), jnp.int32)
    for b in range(BK):
        v = jnp.where(lane == b, nfinal[b], v)
    cntw[pl.ds(0, 16)] = v
    cntw[pl.ds(16, 16)] = jnp.zeros((16,), jnp.int32)
    pltpu.sync_copy(cntw, ngrp_hbm.at[t])


@functools.partial(
    pl.kernel, mesh=_mesh,
    out_type=jax.ShapeDtypeStruct((BK, BS, H), jnp.float32),
    scratch_types=[
        pltpu.VMEM((4, 128), jnp.int32),
        pltpu.VMEM((4, 128), jnp.int32),
        pltpu.VMEM((4, 128, H), jnp.float32),
        pltpu.VMEM((128, H), jnp.float32),
        pltpu.VMEM((64,), jnp.int32),
        pltpu.VMEM_SHARED((ACC_R, H), jnp.float32),
        pltpu.SemaphoreType.DMA,
    ],
    compiler_params=_sc_params,
)
def _acc_kernel(zeros_hbm, y_hbm, bsrc_hbm, bdst_hbm, ngrp_hbm, out_hbm,
                srcbuf, dstbuf, rows, zbuf, cntv, acc, sem):
    cid = lax.axis_index("c")
    sid = lax.axis_index("s")

    pltpu.sync_copy(zeros_hbm, zbuf)
    pltpu.sync_copy(ngrp_hbm.at[2 * sid], cntv.at[pl.ds(0, 32)])
    pltpu.sync_copy(ngrp_hbm.at[2 * sid + 1], cntv.at[pl.ds(32, 32)])

    for bo in range(4):
        b = cid * 4 + bo
        for z in range(7):
            pltpu.sync_copy(zbuf, acc.at[pl.ds((sid * 7 + z) * 128, 128)])
        plsc.subcore_barrier()

        for p in range(2):
            t = 2 * sid + p
            ng = cntv[pl.ds(32 * p + b, 16)][0]
            def grp_body(g, _):
                pltpu.sync_copy(bsrc_hbm.at[t].at[b].at[g], srcbuf)
                pltpu.sync_copy(bdst_hbm.at[t].at[b].at[g], dstbuf)
                hs = [pltpu.async_copy(y_hbm.at[srcbuf.at[j]], rows.at[j], sem)
                      for j in range(4)]
                for h in hs:
                    h.wait()
                for j in range(4):
                    pltpu.sync_copy(rows.at[j], acc.at[dstbuf.at[j]], add=True)
                return 0
            lax.fori_loop(0, ng, grp_body, 0)
        plsc.subcore_barrier()
        pltpu.sync_copy(acc.at[pl.ds(sid * 784, 784)],
                        out_hbm.at[b].at[pl.ds(sid * 784, 784)])
        plsc.subcore_barrier()


PN = 102400


@functools.partial(
    pl.kernel, mesh=_mesh,
    out_type=(
        jax.ShapeDtypeStruct((2, 1040, H), jnp.float32),
        jax.ShapeDtypeStruct((2, 1040, H), jnp.float32),
    ),
    scratch_types=[
        pltpu.VMEM((128,), jnp.int32),
        pltpu.VMEM((1, 128), jnp.int32),
        pltpu.VMEM((128, H), jnp.float32),
        pltpu.VMEM((128, H), jnp.float32),
        pltpu.VMEM_SHARED((1040, H), jnp.float32),
        pltpu.VMEM_SHARED((1040, H), jnp.float32),
    ],
    compiler_params=_sc_params,
)
def _pool_kernel(zeros_hbm, ones_hbm, h_hbm, bid_hbm, ridx_hbm,
                 spool, cpool, idxg, bidb, hbuf, onesb, accs, accc):
    cid = lax.axis_index("c")
    sid = lax.axis_index("s")
    wid = cid * 16 + sid

    pltpu.sync_copy(ones_hbm, onesb)
    pltpu.sync_copy(zeros_hbm.at[pl.ds(0, 65)], accs.at[pl.ds(sid * 65, 65)])
    pltpu.sync_copy(zeros_hbm.at[pl.ds(0, 65)], accc.at[pl.ds(sid * 65, 65)])
    plsc.subcore_barrier()

    def chunk_body(c, _):
        pltpu.sync_copy(ridx_hbm.at[wid].at[c], idxg)
        pltpu.sync_copy(bid_hbm.at[pl.ds(wid * 3200 + c * 128, 128)],
                        bidb.at[0])
        pltpu.sync_copy(h_hbm.at[idxg], hbuf)
        pltpu.sync_copy(hbuf, accs.at[bidb.at[0]], add=True)
        pltpu.sync_copy(onesb, accc.at[bidb.at[0]], add=True)
        return 0
    lax.fori_loop(0, 25, chunk_body, 0)
    plsc.subcore_barrier()
    pltpu.sync_copy(accs.at[pl.ds(sid * 65, 65)],
                    spool.at[cid].at[pl.ds(sid * 65, 65)])
    pltpu.sync_copy(accc.at[pl.ds(sid * 65, 65)],
                    cpool.at[cid].at[pl.ds(sid * 65, 65)])


def _degsum_body(dp_ref, o_ref):
    s = jnp.sum(dp_ref[...], axis=0) + 1.0
    o_ref[...] = lax.rsqrt(s)


def _degsum(degp):
    return pl.pallas_call(
        _degsum_body,
        out_shape=jax.ShapeDtypeStruct((NP2,), jnp.float32),
        grid=(NP2 // 1024,),
        in_specs=[pl.BlockSpec((32, 1024), lambda j: (0, j))],
        out_specs=pl.BlockSpec((1024,), lambda j: (j,)),
    )(degp)


def _prep_body(x_ref, w_ref, dv_ref, o_ref):
    xw = jnp.dot(x_ref[...], w_ref[...], preferred_element_type=jnp.float32)
    o_ref[...] = xw * dv_ref[...]


def _prep(x, W1, dinv2d):
    blk = 3136
    return pl.pallas_call(
        _prep_body,
        out_shape=jax.ShapeDtypeStruct((N, H), jnp.float32),
        grid=(32,),
        in_specs=[
            pl.BlockSpec((blk, 9), lambda j: (j, 0)),
            pl.BlockSpec((9, H), lambda j: (0, 0)),
            pl.BlockSpec((blk, 1), lambda j: (j, 0)),
        ],
        out_specs=pl.BlockSpec((blk, H), lambda j: (j, 0)),
    )(x, W1, dinv2d)


def _post_body(acc_ref, y_ref, dv_ref, b_ref, w_ref, o_ref):
    dv = dv_ref[...]
    h = jnp.maximum((acc_ref[0] + y_ref[...]) * dv + b_ref[...], 0.0)
    o_ref[...] = jnp.dot(h, w_ref[...],
                         preferred_element_type=jnp.float32) * dv


def _post3_body(acc_ref, y_ref, dv_ref, b_ref, o_ref):
    dv = dv_ref[...]
    o_ref[...] = jnp.maximum((acc_ref[0] + y_ref[...]) * dv + b_ref[...], 0.0)


def _post(acc, y, dinv2d, bvec, Wn):
    blk = 3136
    args = [acc, y, dinv2d, bvec.reshape(1, H)]
    in_specs = [
        pl.BlockSpec((1, blk, H), lambda b, j: (b, j, 0)),
        pl.BlockSpec((blk, H), lambda b, j: (4 * b + j, 0)),
        pl.BlockSpec((blk, 1), lambda b, j: (4 * b + j, 0)),
        pl.BlockSpec((1, H), lambda b, j: (0, 0)),
    ]
    if Wn is not None:
        args.append(Wn)
        in_specs.append(pl.BlockSpec((H, H), lambda b, j: (0, 0)))
        body = _post_body
    else:
        body = _post3_body
    return pl.pallas_call(
        body,
        out_shape=jax.ShapeDtypeStruct((N, H), jnp.float32),
        grid=(BK, 4),
        in_specs=in_specs,
        out_specs=pl.BlockSpec((blk, H), lambda b, j: (4 * b + j, 0)),
    )(*args)


def _final_body(s_ref, c_ref, w_ref, b_ref, o_ref):
    s = s_ref[0] + s_ref[1]
    c = jnp.maximum(c_ref[0] + c_ref[1], 1.0)
    pooled = s / c
    o_ref[...] = jnp.dot(pooled, w_ref[...],
                         preferred_element_type=jnp.float32) + b_ref[0, 0]


def _final(spool, cpool, Wl, bl):
    return pl.pallas_call(
        _final_body,
        out_shape=jax.ShapeDtypeStruct((G, 1), jnp.float32),
        in_specs=[
            pl.BlockSpec((2, G, H), lambda: (0, 0, 0)),
            pl.BlockSpec((2, G, H), lambda: (0, 0, 0)),
            pl.BlockSpec((H, 1), lambda: (0, 0)),
            pl.BlockSpec((1, 1), lambda: (0, 0)),
        ],
        out_specs=pl.BlockSpec((G, 1), lambda: (0, 0)),
    )(spool[:, :G, :], cpool[:, :G, :], Wl, bl.reshape(1, 1))


def kernel(x, edge_index, batch, W1, b1, W2, b2, W3, b3, Wl, bl):
    src = edge_index[0]
    dst = edge_index[1]

    degp = _deg_kernel(dst)
    bsrc, bdst, ngrp = _bin_kernel(src, dst)

    dinv1 = _degsum(degp)
    dinv2d = dinv1[:N].reshape(N, 1)

    zeros = jnp.zeros((128, H), jnp.float32)
    ones = jnp.ones((128, H), jnp.float32)

    y1 = _prep(x, W1, dinv2d)
    acc1 = _acc_kernel(zeros, y1, bsrc, bdst, ngrp)
    y2 = _post(acc1, y1, dinv2d, b1, W2)
    acc2 = _acc_kernel(zeros, y2, bsrc, bdst, ngrp)
    y3 = _post(acc2, y2, dinv2d, b2, W3)
    acc3 = _acc_kernel(zeros, y3, bsrc, bdst, ngrp)
    h3 = _post(acc3, y3, dinv2d, b3, None)

    bid_pad = jnp.concatenate(
        [batch, jnp.full((PN - N,), G, jnp.int32)])
    ridx = jnp.minimum(jnp.arange(PN, dtype=jnp.int32), N - 1)
    ridx = ridx.reshape(32, 25, 128)
    spool, cpool = _pool_kernel(zeros, ones, h3, bid_pad, ridx)
    return _final(spool, cpool, Wl, bl)


# speedup vs baseline: 7.8389x; 1.0521x over previous
import functools

import jax
import jax.numpy as jnp
from jax import lax
from jax.experimental import pallas as pl
from jax.experimental.pallas import tpu as pltpu
from jax.experimental.pallas import tpu_sc as plsc

N = 100000
E = 1600000
H = 64
G = 1024

NP2 = 100352
BK = 8
BS = 12544
SAC = BS
ACC_R = 12800
PT_E = E // 32
GRP = 512
MAXG = (PT_E + GRP - 1) // GRP
SL = NP2 // 16
CH = 10000

_mesh = plsc.VectorSubcoreMesh(core_axis_name="c", subcore_axis_name="s")
_sc_params = pltpu.CompilerParams(
    needs_layout_passes=False, use_tc_tiling_on_sc=False)


@functools.partial(
    pl.kernel, mesh=_mesh,
    out_type=jax.ShapeDtypeStruct((32, NP2), jnp.float32),
    scratch_types=[
        pltpu.VMEM((NP2,), jnp.float32),
        pltpu.VMEM((CH,), jnp.int32),
    ],
    compiler_params=_sc_params,
)
def _deg_kernel(dst_hbm, out_hbm, partial, dstbuf):
    cid = lax.axis_index("c")
    sid = lax.axis_index("s")
    wid = cid * 16 + sid

    zeros = jnp.zeros((16,), jnp.float32)
    def zbody(i, _):
        partial[pl.ds(i * 16, 16)] = zeros
        return 0
    lax.fori_loop(0, NP2 // 16, zbody, 0)

    ones = jnp.ones((16,), jnp.float32)
    ebase = wid * PT_E
    def chunk_body(k, _):
        pltpu.sync_copy(dst_hbm.at[pl.ds(ebase + k * CH, CH)], dstbuf)
        def vbody(i, _):
            idx = dstbuf[pl.ds(i * 16, 16)]
            plsc.addupdate_scatter(partial, [idx], ones)
            return 0
        lax.fori_loop(0, CH // 16, vbody, 0)
        return 0
    lax.fori_loop(0, PT_E // CH, chunk_body, 0)
    pltpu.sync_copy(partial, out_hbm.at[wid])


@functools.partial(
    pl.kernel, mesh=_mesh,
    out_type=(
        jax.ShapeDtypeStruct((32, BK, MAXG, 4, 128), jnp.int32),
        jax.ShapeDtypeStruct((32, BK, MAXG, 4, 128), jnp.int32),
        jax.ShapeDtypeStruct((32, 32), jnp.int32),
    ),
    scratch_types=[
        pltpu.VMEM((CH,), jnp.int32),
        pltpu.VMEM((CH,), jnp.int32),
        pltpu.VMEM((BK * 1024,), jnp.int32),
        pltpu.VMEM((BK * 1024,), jnp.int32),
        pltpu.VMEM((32,), jnp.int32),
    ],
    compiler_params=_sc_params,
)
def _bin_kernel(src_hbm, dst_hbm, bsrc_hbm, bdst_hbm, ngrp_hbm,
                sbuf, dbuf, sstg, dstg, cntw):
    cid = lax.axis_index("c")
    sid = lax.axis_index("s")
    t = cid * 16 + sid
    ebase = t * PT_E

    def flush(b, g):
        for j in range(4):
            pltpu.sync_copy(sstg.at[pl.ds(b * 1024 + j * 128, 128)],
                            bsrc_hbm.at[t].at[b].at[g].at[j])
            pltpu.sync_copy(dstg.at[pl.ds(b * 1024 + j * 128, 128)],
                            bdst_hbm.at[t].at[b].at[g].at[j])

    def chunk_body(c, carry):
        pltpu.sync_copy(src_hbm.at[pl.ds(ebase + c * CH, CH)], sbuf)
        pltpu.sync_copy(dst_hbm.at[pl.ds(ebase + c * CH, CH)], dbuf)

        def sub_body(s, carry):
            def vreg_body(v, carry):
                fills, ngs = carry
                off = (s * 25 + v) * 16
                sv = sbuf[pl.ds(off, 16)]
                dv = dbuf[pl.ds(off, 16)]
                nf = []
                for b in range(BK):
                    msk = (dv >= b * BS) & (dv < (b + 1) * BS)
                    plsc.store_compressed(
                        sstg.at[pl.ds(b * 1024 + fills[b], 16)], sv, mask=msk)
                    plsc.store_compressed(
                        dstg.at[pl.ds(b * 1024 + fills[b], 16)],
                        dv - b * BS, mask=msk)
                    cnt = plsc.all_reduce_population_count(msk)[0]
                    nf.append(fills[b] + cnt)
                return tuple(nf), ngs
            carry = lax.fori_loop(0, 25, vreg_body, carry)
            fills, ngs = carry
            nf, nn = [], []
            for b in range(BK):
                def do_flush(fb, gb, b=b):
                    flush(b, gb)
                    for k in range(25):
                        sstg[pl.ds(b * 1024 + k * 16, 16)] = (
                            sstg[pl.ds(b * 1024 + 512 + k * 16, 16)])
                        dstg[pl.ds(b * 1024 + k * 16, 16)] = (
                            dstg[pl.ds(b * 1024 + 512 + k * 16, 16)])
                    return fb - 512, gb + 1
                fb, gb = lax.cond(
                    fills[b] >= 512, do_flush,
                    lambda fb, gb: (fb, gb), fills[b], ngs[b])
                nf.append(fb)
                nn.append(gb)
            return tuple(nf), tuple(nn)
        return lax.fori_loop(0, 25, sub_body, carry)

    z8 = (jnp.int32(0),) * BK
    fills, ngs = lax.fori_loop(0, PT_E // CH, chunk_body, (z8, z8))

    zv = jnp.zeros((16,), jnp.int32)
    sacv = jnp.full((16,), SAC, jnp.int32)
    nfinal = []
    for b in range(BK):
        for k in range(32):
            sstg[pl.ds(b * 1024 + fills[b] + k * 16, 16)] = zv
            dstg[pl.ds(b * 1024 + fills[b] + k * 16, 16)] = sacv
        def do_final(gb, b=b):
            flush(b, gb)
            return gb + 1
        gb = lax.cond(fills[b] > 0, do_final, lambda gb: gb, ngs[b])
        nfinal.append(gb)

    lane = lax.iota(jnp.int32, 16)
    v = jnp.zeros((16,), jnp.int32)
    for b in range(BK):
        v = jnp.where(lane == b, nfinal[b], v)
    cntw[pl.ds(0, 16)] = v
    cntw[pl.ds(16, 16)] = jnp.zeros((16,), jnp.int32)
    pltpu.sync_copy(cntw, ngrp_hbm.at[t])


@functools.partial(
    pl.kernel, mesh=_mesh,
    out_type=jax.ShapeDtypeStruct((BK, BS, H), jnp.float32),
    scratch_types=[
        pltpu.VMEM((4, 4, 128), jnp.int32),
        pltpu.VMEM((4, 4, 128), jnp.int32),
        pltpu.VMEM((2, 4, 128, H), jnp.float32),
        pltpu.VMEM((128, H), jnp.float32),
        pltpu.VMEM((64,), jnp.int32),
        pltpu.VMEM_SHARED((ACC_R, H), jnp.float32),
        pltpu.SemaphoreType.DMA,
        pltpu.SemaphoreType.DMA,
        pltpu.SemaphoreType.DMA,
        pltpu.SemaphoreType.DMA,
        pltpu.SemaphoreType.DMA,
        pltpu.SemaphoreType.DMA,
        pltpu.SemaphoreType.DMA,
        pltpu.SemaphoreType.DMA,
    ],
    compiler_params=_sc_params,
)
def _acc_kernel(zeros_hbm, y_hbm, bsrc_hbm, bdst_hbm, ngrp_hbm, out_hbm,
                srcbuf, dstbuf, rows, zbuf, cntv, acc,
                si0, si1, si2, si3, sg0, sg1, ss0, ss1):
    cid = lax.axis_index("c")
    sid = lax.axis_index("s")
    semi = [si0, si1, si2, si3]
    semg = [sg0, sg1]
    sems = [ss0, ss1]

    pltpu.sync_copy(zeros_hbm, zbuf)
    pltpu.sync_copy(ngrp_hbm.at[2 * sid], cntv.at[pl.ds(0, 32)])
    pltpu.sync_copy(ngrp_hbm.at[2 * sid + 1], cntv.at[pl.ds(32, 32)])

    def fire_idx(t, b, g, s):
        pltpu.async_copy(bsrc_hbm.at[t].at[b].at[g], srcbuf.at[s], semi[s])
        pltpu.async_copy(bdst_hbm.at[t].at[b].at[g], dstbuf.at[s], semi[s])

    def wait_idx(s):
        pltpu.make_async_copy(
            bsrc_hbm.at[0].at[0].at[0], srcbuf.at[s], semi[s]).wait()
        pltpu.make_async_copy(
            bdst_hbm.at[0].at[0].at[0], dstbuf.at[s], semi[s]).wait()

    def drain_scat(bf):
        for j in range(4):
            pltpu.make_async_copy(
                rows.at[bf].at[j], acc.at[dstbuf.at[0].at[j]],
                sems[bf]).wait()

    for bo in range(4):
        b = cid * 4 + bo
        for z in range(6):
            pltpu.sync_copy(zbuf, acc.at[pl.ds(sid * 800 + z * 128, 128)])
        pltpu.sync_copy(zbuf.at[pl.ds(0, 32)],
                        acc.at[pl.ds(sid * 800 + 768, 32)])
        plsc.subcore_barrier()

        for p in range(2):
            t = 2 * sid + p
            ng = cntv[pl.ds(32 * p + b, 16)][0]
            for s in range(2):
                @pl.when(s < ng)
                def _(s=s):
                    fire_idx(t, b, s, s)

            def quad(i, _):
                for q in range(4):
                    bf = q % 2
                    g = 4 * i + q
                    @pl.when(g < ng)
                    def _(g=g, q=q, bf=bf):
                        @pl.when(g >= 2)
                        def _():
                            drain_scat(bf)
                        @pl.when(g + 2 < ng)
                        def _():
                            fire_idx(t, b, g + 2, (q + 2) % 4)
                        wait_idx(q)
                        hs = [pltpu.async_copy(
                            y_hbm.at[srcbuf.at[q].at[j]],
                            rows.at[bf].at[j], semg[bf]) for j in range(4)]
                        for h in hs:
                            h.wait()
                        for j in range(4):
                            pltpu.async_copy(
                                rows.at[bf].at[j],
                                acc.at[dstbuf.at[q].at[j]], sems[bf],
                                add=True)
                return 0
            lax.fori_loop(0, (ng + 3) // 4, quad, 0)
            @pl.when(ng >= 1)
            def _():
                drain_scat(0)
            @pl.when(ng >= 2)
            def _():
                drain_scat(1)
        plsc.subcore_barrier()
        pltpu.sync_copy(acc.at[pl.ds(sid * 784, 784)],
                        out_hbm.at[b].at[pl.ds(sid * 784, 784)])
        plsc.subcore_barrier()


PN = 102400


@functools.partial(
    pl.kernel, mesh=_mesh,
    out_type=(
        jax.ShapeDtypeStruct((2, 1040, H), jnp.float32),
        jax.ShapeDtypeStruct((2, 1040, H), jnp.float32),
    ),
    scratch_types=[
        pltpu.VMEM((128,), jnp.int32),
        pltpu.VMEM((1, 128), jnp.int32),
        pltpu.VMEM((128, H), jnp.float32),
        pltpu.VMEM((128, H), jnp.float32),
        pltpu.VMEM_SHARED((1040, H), jnp.float32),
        pltpu.VMEM_SHARED((1040, H), jnp.float32),
    ],
    compiler_params=_sc_params,
)
def _pool_kernel(zeros_hbm, ones_hbm, h_hbm, bid_hbm, ridx_hbm,
                 spool, cpool, idxg, bidb, hbuf, onesb, accs, accc):
    cid = lax.axis_index("c")
    sid = lax.axis_index("s")
    wid = cid * 16 + sid

    pltpu.sync_copy(ones_hbm, onesb)
    pltpu.sync_copy(zeros_hbm.at[pl.ds(0, 65)], accs.at[pl.ds(sid * 65, 65)])
    pltpu.sync_copy(zeros_hbm.at[pl.ds(0, 65)], accc.at[pl.ds(sid * 65, 65)])
    plsc.subcore_barrier()

    def chunk_body(c, _):
        pltpu.sync_copy(ridx_hbm.at[wid].at[c], idxg)
        pltpu.sync_copy(bid_hbm.at[pl.ds(wid * 3200 + c * 128, 128)],
                        bidb.at[0])
        pltpu.sync_copy(h_hbm.at[idxg], hbuf)
        pltpu.sync_copy(hbuf, accs.at[bidb.at[0]], add=True)
        pltpu.sync_copy(onesb, accc.at[bidb.at[0]], add=True)
        return 0
    lax.fori_loop(0, 25, chunk_body, 0)
    plsc.subcore_barrier()
    pltpu.sync_copy(accs.at[pl.ds(sid * 65, 65)],
                    spool.at[cid].at[pl.ds(sid * 65, 65)])
    pltpu.sync_copy(accc.at[pl.ds(sid * 65, 65)],
                    cpool.at[cid].at[pl.ds(sid * 65, 65)])


def _degsum_body(dp_ref, o_ref):
    s = jnp.sum(dp_ref[...], axis=0) + 1.0
    o_ref[...] = lax.rsqrt(s)


def _degsum(degp):
    return pl.pallas_call(
        _degsum_body,
        out_shape=jax.ShapeDtypeStruct((NP2,), jnp.float32),
        grid=(NP2 // 1024,),
        in_specs=[pl.BlockSpec((32, 1024), lambda j: (0, j))],
        out_specs=pl.BlockSpec((1024,), lambda j: (j,)),
    )(degp)


def _prep_body(x_ref, w_ref, dv_ref, o_ref):
    xw = jnp.dot(x_ref[...], w_ref[...], preferred_element_type=jnp.float32)
    o_ref[...] = xw * dv_ref[...]


def _prep(x, W1, dinv2d):
    blk = 3136
    return pl.pallas_call(
        _prep_body,
        out_shape=jax.ShapeDtypeStruct((N, H), jnp.float32),
        grid=(32,),
        in_specs=[
            pl.BlockSpec((blk, 9), lambda j: (j, 0)),
            pl.BlockSpec((9, H), lambda j: (0, 0)),
            pl.BlockSpec((blk, 1), lambda j: (j, 0)),
        ],
        out_specs=pl.BlockSpec((blk, H), lambda j: (j, 0)),
    )(x, W1, dinv2d)


def _post_body(acc_ref, y_ref, dv_ref, b_ref, w_ref, o_ref):
    dv = dv_ref[...]
    h = jnp.maximum((acc_ref[0] + y_ref[...]) * dv + b_ref[...], 0.0)
    o_ref[...] = jnp.dot(h, w_ref[...],
                         preferred_element_type=jnp.float32) * dv


def _post3_body(acc_ref, y_ref, dv_ref, b_ref, o_ref):
    dv = dv_ref[...]
    o_ref[...] = jnp.maximum((acc_ref[0] + y_ref[...]) * dv + b_ref[...], 0.0)


def _post(acc, y, dinv2d, bvec, Wn):
    blk = 3136
    args = [acc, y, dinv2d, bvec.reshape(1, H)]
    in_specs = [
        pl.BlockSpec((1, blk, H), lambda b, j: (b, j, 0)),
        pl.BlockSpec((blk, H), lambda b, j: (4 * b + j, 0)),
        pl.BlockSpec((blk, 1), lambda b, j: (4 * b + j, 0)),
        pl.BlockSpec((1, H), lambda b, j: (0, 0)),
    ]
    if Wn is not None:
        args.append(Wn)
        in_specs.append(pl.BlockSpec((H, H), lambda b, j: (0, 0)))
        body = _post_body
    else:
        body = _post3_body
    return pl.pallas_call(
        body,
        out_shape=jax.ShapeDtypeStruct((N, H), jnp.float32),
        grid=(BK, 4),
        in_specs=in_specs,
        out_specs=pl.BlockSpec((blk, H), lambda b, j: (4 * b + j, 0)),
    )(*args)


def _final_body(s_ref, c_ref, w_ref, b_ref, o_ref):
    s = s_ref[0] + s_ref[1]
    c = jnp.maximum(c_ref[0] + c_ref[1], 1.0)
    pooled = s / c
    o_ref[...] = jnp.dot(pooled, w_ref[...],
                         preferred_element_type=jnp.float32) + b_ref[0, 0]


def _final(spool, cpool, Wl, bl):
    return pl.pallas_call(
        _final_body,
        out_shape=jax.ShapeDtypeStruct((G, 1), jnp.float32),
        in_specs=[
            pl.BlockSpec((2, G, H), lambda: (0, 0, 0)),
            pl.BlockSpec((2, G, H), lambda: (0, 0, 0)),
            pl.BlockSpec((H, 1), lambda: (0, 0)),
            pl.BlockSpec((1, 1), lambda: (0, 0)),
        ],
        out_specs=pl.BlockSpec((G, 1), lambda: (0, 0)),
    )(spool[:, :G, :], cpool[:, :G, :], Wl, bl.reshape(1, 1))


def kernel(x, edge_index, batch, W1, b1, W2, b2, W3, b3, Wl, bl):
    src = edge_index[0]
    dst = edge_index[1]

    degp = _deg_kernel(dst)
    bsrc, bdst, ngrp = _bin_kernel(src, dst)

    dinv1 = _degsum(degp)
    dinv2d = dinv1[:N].reshape(N, 1)

    zeros = jnp.zeros((128, H), jnp.float32)
    ones = jnp.ones((128, H), jnp.float32)

    y1 = _prep(x, W1, dinv2d)
    acc1 = _acc_kernel(zeros, y1, bsrc, bdst, ngrp)
    y2 = _post(acc1, y1, dinv2d, b1, W2)
    acc2 = _acc_kernel(zeros, y2, bsrc, bdst, ngrp)
    y3 = _post(acc2, y2, dinv2d, b2, W3)
    acc3 = _acc_kernel(zeros, y3, bsrc, bdst, ngrp)
    h3 = _post(acc3, y3, dinv2d, b3, None)

    bid_pad = jnp.concatenate(
        [batch, jnp.full((PN - N,), G, jnp.int32)])
    ridx = jnp.minimum(jnp.arange(PN, dtype=jnp.int32), N - 1)
    ridx = ridx.reshape(32, 25, 128)
    spool, cpool = _pool_kernel(zeros, ones, h3, bid_pad, ridx)
    return _final(spool, cpool, Wl, bl)
